# baseline (speedup 1.0000x reference)
import functools

import jax
import jax.numpy as jnp
from jax import lax
from jax.experimental import pallas as pl
from jax.experimental.pallas import tpu as pltpu
from jax.experimental.pallas import tpu_sc as plsc

_NC = 2
_NS = 16
_L = 16
_CH = 256

_DO_SCALE = True
_DO_GATHER = True
_DO_SCATTER = True


def _sc_aggregate(x, src, dst, adj):
    N, D = x.shape
    NW = _NC * _NS
    nk = src.shape[0] // (NW * _CH)
    nj = D // _L
    rpt = (N // (_NS * 8)) * 8
    left = N - _NS * rpt
    zfull = rpt // _CH
    zrem = rpt - zfull * _CH

    mesh = plsc.VectorSubcoreMesh(core_axis_name="c", subcore_axis_name="s")

    scratch = [
        pltpu.VMEM((_CH,), jnp.int32),
        pltpu.VMEM((_CH,), jnp.int32),
        pltpu.VMEM((_CH,), jnp.float32),
        pltpu.VMEM((_CH, D), jnp.float32),
        pltpu.VMEM_SHARED((N, D), jnp.float32),
        pltpu.SemaphoreType.DMA,
    ]

    @functools.partial(
        pl.kernel,
        out_type=jax.ShapeDtypeStruct((_NC, N, D), jnp.float32),
        mesh=mesh,
        scratch_types=scratch,
    )
    def agg(x_hbm, src_hbm, dst_hbm, adj_hbm, out_hbm, *refs):
        srcv, dstv, adjv, msg, acc, sem = refs
        c = lax.axis_index("c")
        s = lax.axis_index("s")
        wid = c * _NS + s

        zero = jnp.zeros((_L,), jnp.float32)

        def zrow(r, carry):
            for j in range(nj):
                msg[r, pl.ds(j * _L, _L)] = zero
            return carry

        lax.fori_loop(0, _CH, zrow, 0)
        r0 = s * rpt
        for i in range(zfull):
            pltpu.sync_copy(msg, acc.at[pl.ds(r0 + i * _CH, _CH)])
        if zrem:
            pltpu.sync_copy(msg.at[pl.ds(0, zrem)],
                            acc.at[pl.ds(r0 + zfull * _CH, zrem)])
        if left:
            @pl.when(s == 0)
            def _():
                pltpu.sync_copy(msg.at[pl.ds(0, left)],
                                acc.at[pl.ds(_NS * rpt, left)])
        plsc.subcore_barrier()

        def scale_rows(msg_ref, adj_ref, n):
            def grp(g, carry):
                a16 = adj_ref[pl.ds(g * _L, _L)]
                for r in range(_L):
                    av = lax.broadcast(a16[r], (_L,))
                    row = g * _L + r
                    for j in range(nj):
                        sl = pl.ds(j * _L, _L)
                        msg_ref[row, sl] = msg_ref[row, sl] * av
                return carry

            lax.fori_loop(0, n // _L, grp, 0)

        e0 = wid * nk * _CH

        def chunk(k, carry):
            base = e0 + k * _CH
            pltpu.sync_copy(src_hbm.at[pl.ds(base, _CH)], srcv)
            pltpu.sync_copy(dst_hbm.at[pl.ds(base, _CH)], dstv)
            pltpu.sync_copy(adj_hbm.at[pl.ds(base, _CH)], adjv)
            if _DO_GATHER:
                pltpu.async_copy(x_hbm.at[srcv], msg, sem).wait()
            if _DO_SCALE:
                scale_rows(msg, adjv, _CH)
            if _DO_SCATTER:
                pltpu.sync_copy(msg, acc.at[dstv], add=True)
            return carry

        lax.fori_loop(0, nk, chunk, 0)
        plsc.subcore_barrier()

        for i in range(zfull):
            sl = pl.ds(r0 + i * _CH, _CH)
            pltpu.sync_copy(acc.at[sl], out_hbm.at[c, sl])
        if zrem:
            sl = pl.ds(r0 + zfull * _CH, zrem)
            pltpu.sync_copy(acc.at[sl], out_hbm.at[c, sl])
        if left:
            @pl.when(s == 0)
            def _():
                sl = pl.ds(_NS * rpt, left)
                pltpu.sync_copy(acc.at[sl], out_hbm.at[c, sl])

    return agg(x, src, dst, adj)


def _tc_finish(partials, W):
    _, N, D = partials.shape
    blk = 1000

    def body(p_ref, w_ref, o_ref):
        acc = p_ref[0] + p_ref[1]
        h = jnp.dot(acc, w_ref[...], preferred_element_type=jnp.float32)
        o_ref[...] = jnp.maximum(h, 0.0)

    return pl.pallas_call(
        body,
        grid=(N // blk,),
        in_specs=[
            pl.BlockSpec((2, blk, D), lambda i: (0, i, 0)),
            pl.BlockSpec((D, D), lambda i: (0, 0)),
        ],
        out_specs=pl.BlockSpec((blk, D), lambda i: (i, 0)),
        out_shape=jax.ShapeDtypeStruct((N, D), jnp.float32),
    )(partials, W)


def kernel(x, edge_index, adj_values, W):
    E = edge_index.shape[1]
    NW = _NC * _NS
    nk = -(-E // (NW * _CH))
    Ep = NW * nk * _CH
    pad = Ep - E
    src = jnp.pad(edge_index[0], (0, pad))
    dst = jnp.pad(edge_index[1], (0, pad))
    adj = jnp.pad(adj_values, (0, pad))
    partials = _sc_aggregate(x, src, dst, adj)
    return _tc_finish(partials, W)


# speedup vs baseline: 1.9661x; 1.9661x over previous
import functools

import jax
import jax.numpy as jnp
from jax import lax
from jax.experimental import pallas as pl
from jax.experimental.pallas import tpu as pltpu
from jax.experimental.pallas import tpu_sc as plsc


def _sc_aggregate(x, src, dst, adj):
    N, D = x.shape
    E = src.shape[0]
    _NC, _NS, _L, _CH = 2, 16, 16, 128
    NW = _NC * _NS
    eb = E // NW
    nfull = eb // _CH
    tail = eb - nfull * _CH
    nj = D // _L
    rpt = (N // (_NS * 8)) * 8
    left = N - _NS * rpt
    zfull = rpt // _CH
    zrem = rpt - zfull * _CH

    mesh = plsc.VectorSubcoreMesh(core_axis_name="c", subcore_axis_name="s")

    scratch = [
        pltpu.VMEM((_CH,), jnp.int32),
        pltpu.VMEM((_CH,), jnp.int32),
        pltpu.VMEM((_CH,), jnp.float32),
        pltpu.VMEM((_CH, D), jnp.float32),
        pltpu.VMEM_SHARED((N, D), jnp.float32),
        pltpu.SemaphoreType.DMA,
    ]
    if tail:
        scratch += [
            pltpu.VMEM((tail,), jnp.int32),
            pltpu.VMEM((tail,), jnp.int32),
            pltpu.VMEM((tail,), jnp.float32),
            pltpu.VMEM((tail, D), jnp.float32),
        ]

    @functools.partial(
        pl.kernel,
        out_type=jax.ShapeDtypeStruct((_NC, N, D), jnp.float32),
        mesh=mesh,
        scratch_types=scratch,
    )
    def agg(x_hbm, src_hbm, dst_hbm, adj_hbm, out_hbm, *refs):
        if tail:
            srcv, dstv, adjv, msg, acc, sem, srct, dstt, adjt, msgt = refs
        else:
            srcv, dstv, adjv, msg, acc, sem = refs
        c = lax.axis_index("c")
        s = lax.axis_index("s")
        wid = c * _NS + s

        zero = jnp.zeros((_L,), jnp.float32)

        def zrow(r, carry):
            for j in range(nj):
                msg[r, pl.ds(j * _L, _L)] = zero
            return carry

        lax.fori_loop(0, _CH, zrow, 0)
        r0 = s * rpt
        for i in range(zfull):
            pltpu.sync_copy(msg, acc.at[pl.ds(r0 + i * _CH, _CH)])
        if zrem:
            pltpu.sync_copy(msg.at[pl.ds(0, zrem)],
                            acc.at[pl.ds(r0 + zfull * _CH, zrem)])
        if left:
            @pl.when(s == 0)
            def _():
                pltpu.sync_copy(msg.at[pl.ds(0, left)],
                                acc.at[pl.ds(_NS * rpt, left)])
        plsc.subcore_barrier()

        def scale_rows(msg_ref, adj_ref, n):
            def grp(g, carry):
                a16 = adj_ref[pl.ds(g * _L, _L)]
                for r in range(_L):
                    av = lax.broadcast(a16[r], (_L,))
                    row = g * _L + r
                    for j in range(nj):
                        sl = pl.ds(j * _L, _L)
                        msg_ref[row, sl] = msg_ref[row, sl] * av
                return carry

            lax.fori_loop(0, n // _L, grp, 0)

        e0 = wid * eb

        def chunk(k, carry):
            base = e0 + k * _CH
            pltpu.sync_copy(src_hbm.at[pl.ds(base, _CH)], srcv)
            pltpu.sync_copy(dst_hbm.at[pl.ds(base, _CH)], dstv)
            pltpu.sync_copy(adj_hbm.at[pl.ds(base, _CH)], adjv)
            pltpu.async_copy(x_hbm.at[srcv], msg, sem).wait()
            scale_rows(msg, adjv, _CH)
            pltpu.sync_copy(msg, acc.at[dstv], add=True)
            return carry

        lax.fori_loop(0, nfull, chunk, 0)

        if tail:
            base = e0 + nfull * _CH
            pltpu.sync_copy(src_hbm.at[pl.ds(base, tail)], srct)
            pltpu.sync_copy(dst_hbm.at[pl.ds(base, tail)], dstt)
            pltpu.sync_copy(adj_hbm.at[pl.ds(base, tail)], adjt)
            pltpu.async_copy(x_hbm.at[srct], msgt, sem).wait()
            scale_rows(msgt, adjt, tail)
            pltpu.sync_copy(msgt, acc.at[dstt], add=True)

        plsc.subcore_barrier()

        for i in range(zfull):
            sl = pl.ds(r0 + i * _CH, _CH)
            pltpu.sync_copy(acc.at[sl], out_hbm.at[c, sl])
        if zrem:
            sl = pl.ds(r0 + zfull * _CH, zrem)
            pltpu.sync_copy(acc.at[sl], out_hbm.at[c, sl])
        if left:
            @pl.when(s == 0)
            def _():
                sl = pl.ds(_NS * rpt, left)
                pltpu.sync_copy(acc.at[sl], out_hbm.at[c, sl])

    return agg(x, src, dst, adj)


def _tc_finish(partials, W):
    _, N, D = partials.shape
    blk = 1000

    def body(p_ref, w_ref, o_ref):
        acc = p_ref[0] + p_ref[1]
        h = jnp.dot(acc, w_ref[...], preferred_element_type=jnp.float32)
        o_ref[...] = jnp.maximum(h, 0.0)

    return pl.pallas_call(
        body,
        grid=(N // blk,),
        in_specs=[
            pl.BlockSpec((2, blk, D), lambda i: (0, i, 0)),
            pl.BlockSpec((D, D), lambda i: (0, 0)),
        ],
        out_specs=pl.BlockSpec((blk, D), lambda i: (i, 0)),
        out_shape=jax.ShapeDtypeStruct((N, D), jnp.float32),
    )(partials, W)


def kernel(x, edge_index, adj_values, W):
    src = edge_index[0]
    dst = edge_index[1]
    partials = _sc_aggregate(x, src, dst, adj_values)
    return _tc_finish(partials, W)


# speedup vs baseline: 3.3597x; 1.7089x over previous
import functools

import jax
import jax.numpy as jnp
from jax import lax
from jax.experimental import pallas as pl
from jax.experimental.pallas import tpu as pltpu
from jax.experimental.pallas import tpu_sc as plsc


def _sc_aggregate(x, src, dst, adj):
    N, D = x.shape
    E = src.shape[0]
    _NC, _NS, _L, _CH = 2, 16, 16, 128
    NW = _NC * _NS
    eb = E // NW
    nfull = eb // _CH
    tail = eb - nfull * _CH
    nj = D // _L
    rpt = (N // (_NS * 8)) * 8
    left = N - _NS * rpt
    zfull = rpt // _CH
    zrem = rpt - zfull * _CH

    mesh = plsc.VectorSubcoreMesh(core_axis_name="c", subcore_axis_name="s")

    scratch = [
        pltpu.VMEM_SHARED((N, D), jnp.float32),
    ]
    scratch += [pltpu.VMEM((_CH,), jnp.int32) for _ in range(2)]
    scratch += [pltpu.VMEM((_CH,), jnp.int32) for _ in range(2)]
    scratch += [pltpu.VMEM((_CH,), jnp.float32) for _ in range(2)]
    scratch += [pltpu.VMEM((_CH, D), jnp.float32) for _ in range(2)]
    scratch += [pltpu.SemaphoreType.DMA for _ in range(4)]
    if tail:
        scratch += [
            pltpu.VMEM((tail,), jnp.int32),
            pltpu.VMEM((tail,), jnp.int32),
            pltpu.VMEM((tail,), jnp.float32),
            pltpu.VMEM((tail, D), jnp.float32),
        ]

    @functools.partial(
        pl.kernel,
        out_type=jax.ShapeDtypeStruct((_NC, N, D), jnp.float32),
        mesh=mesh,
        scratch_types=scratch,
    )
    def agg(x_hbm, src_hbm, dst_hbm, adj_hbm, out_hbm, *refs):
        acc = refs[0]
        srcv = refs[1:3]
        dstv = refs[3:5]
        adjv = refs[5:7]
        msg = refs[7:9]
        isem = refs[9:11]
        gsem = refs[11:13]
        if tail:
            srct, dstt, adjt, msgt = refs[13:17]
        c = lax.axis_index("c")
        s = lax.axis_index("s")
        wid = c * _NS + s

        zero = jnp.zeros((_L,), jnp.float32)

        def zrow(r, carry):
            for j in range(nj):
                msg[0][r, pl.ds(j * _L, _L)] = zero
            return carry

        lax.fori_loop(0, _CH, zrow, 0)
        r0 = s * rpt
        for i in range(zfull):
            pltpu.sync_copy(msg[0], acc.at[pl.ds(r0 + i * _CH, _CH)])
        if zrem:
            pltpu.sync_copy(msg[0].at[pl.ds(0, zrem)],
                            acc.at[pl.ds(r0 + zfull * _CH, zrem)])
        if left:
            @pl.when(s == 0)
            def _():
                pltpu.sync_copy(msg[0].at[pl.ds(0, left)],
                                acc.at[pl.ds(_NS * rpt, left)])
        plsc.subcore_barrier()

        def scale_rows(msg_ref, adj_ref, n):
            def grp(g, carry):
                a16 = adj_ref[pl.ds(g * _L, _L)]
                for r in range(_L):
                    av = lax.broadcast(a16[r], (_L,))
                    row = g * _L + r
                    for j in range(nj):
                        sl = pl.ds(j * _L, _L)
                        msg_ref[row, sl] = msg_ref[row, sl] * av
                return carry

            lax.fori_loop(0, n // _L, grp, 0)

        e0 = wid * eb

        def fetch_idx(k, b):
            base = e0 + k * _CH
            pltpu.async_copy(src_hbm.at[pl.ds(base, _CH)], srcv[b], isem[b])
            pltpu.async_copy(dst_hbm.at[pl.ds(base, _CH)], dstv[b], isem[b])
            pltpu.async_copy(adj_hbm.at[pl.ds(base, _CH)], adjv[b], isem[b])

        def wait_idx(k, b):
            base = e0 + k * _CH
            pltpu.make_async_copy(src_hbm.at[pl.ds(base, _CH)], srcv[b],
                                  isem[b]).wait()
            pltpu.make_async_copy(dst_hbm.at[pl.ds(base, _CH)], dstv[b],
                                  isem[b]).wait()
            pltpu.make_async_copy(adj_hbm.at[pl.ds(base, _CH)], adjv[b],
                                  isem[b]).wait()

        def gather(b):
            pltpu.async_copy(x_hbm.at[srcv[b]], msg[b], gsem[b])

        def wait_gather(b):
            pltpu.make_async_copy(x_hbm.at[srcv[b]], msg[b], gsem[b]).wait()

        fetch_idx(0, 0)
        wait_idx(0, 0)
        gather(0)

        def chunk(g, carry):
            for b in range(2):
                k = 2 * g + b

                @pl.when(k + 1 < nfull)
                def _():
                    fetch_idx(k + 1, 1 - b)

                wait_gather(b)
                scale_rows(msg[b], adjv[b], _CH)

                @pl.when(k + 1 < nfull)
                def _():
                    wait_idx(k + 1, 1 - b)
                    gather(1 - b)

                pltpu.sync_copy(msg[b], acc.at[dstv[b]], add=True)
            return carry

        lax.fori_loop(0, nfull // 2, chunk, 0)

        if tail:
            base = e0 + nfull * _CH
            pltpu.sync_copy(src_hbm.at[pl.ds(base, tail)], srct)
            pltpu.sync_copy(dst_hbm.at[pl.ds(base, tail)], dstt)
            pltpu.sync_copy(adj_hbm.at[pl.ds(base, tail)], adjt)
            pltpu.async_copy(x_hbm.at[srct], msgt, gsem[0]).wait()
            scale_rows(msgt, adjt, tail)
            pltpu.sync_copy(msgt, acc.at[dstt], add=True)

        plsc.subcore_barrier()

        for i in range(zfull):
            sl = pl.ds(r0 + i * _CH, _CH)
            pltpu.sync_copy(acc.at[sl], out_hbm.at[c, sl])
        if zrem:
            sl = pl.ds(r0 + zfull * _CH, zrem)
            pltpu.sync_copy(acc.at[sl], out_hbm.at[c, sl])
        if left:
            @pl.when(s == 0)
            def _():
                sl = pl.ds(_NS * rpt, left)
                pltpu.sync_copy(acc.at[sl], out_hbm.at[c, sl])

    return agg(x, src, dst, adj)


def _tc_finish(partials, W):
    _, N, D = partials.shape
    blk = 1000

    def body(p_ref, w_ref, o_ref):
        acc = p_ref[0] + p_ref[1]
        h = jnp.dot(acc, w_ref[...], preferred_element_type=jnp.float32)
        o_ref[...] = jnp.maximum(h, 0.0)

    return pl.pallas_call(
        body,
        grid=(N // blk,),
        in_specs=[
            pl.BlockSpec((2, blk, D), lambda i: (0, i, 0)),
            pl.BlockSpec((D, D), lambda i: (0, 0)),
        ],
        out_specs=pl.BlockSpec((blk, D), lambda i: (i, 0)),
        out_shape=jax.ShapeDtypeStruct((N, D), jnp.float32),
    )(partials, W)


def kernel(x, edge_index, adj_values, W):
    src = edge_index[0]
    dst = edge_index[1]
    partials = _sc_aggregate(x, src, dst, adj_values)
    return _tc_finish(partials, W)
